# baseline (speedup 1.0000x reference)
import functools

import jax
import jax.numpy as jnp
from jax.experimental import pallas as pl
from jax.experimental.pallas import tpu as pltpu
from jax.experimental.pallas import tpu_sc as plsc


def _router_body(x_ref, gwt_ref, gb_ref, rb_ref, idx_ref, w_ref, *, n_exp):
    x = x_ref[...]
    logits = jnp.dot(x, gwt_ref[...], preferred_element_type=jnp.float32)
    logits = logits + gb_ref[...]
    m = jnp.max(logits, axis=-1, keepdims=True)
    p = jnp.exp(logits - m)
    probas = p / jnp.sum(p, axis=-1, keepdims=True)
    biased = probas + rb_ref[...]
    col = jax.lax.broadcasted_iota(jnp.int32, biased.shape, 1)
    i1 = jnp.argmax(biased, axis=-1, keepdims=True).astype(jnp.int32)
    biased2 = jnp.where(col == i1, -1e30, biased)
    i2 = jnp.argmax(biased2, axis=-1, keepdims=True).astype(jnp.int32)
    p1 = jnp.sum(jnp.where(col == i1, probas, 0.0), axis=-1, keepdims=True)
    p2 = jnp.sum(jnp.where(col == i2, probas, 0.0), axis=-1, keepdims=True)
    tot = p1 + p2
    idx_ref[...] = jnp.concatenate([i1, i2], axis=1)
    w_ref[...] = jnp.concatenate([p1 / tot, p2 / tot], axis=1)


def _router(x2d, gW, gb, router_biases):
    n, emb = x2d.shape
    n_exp = gW.shape[0]
    ep = 128
    tm = 512
    gwt = jnp.zeros((emb, ep), jnp.float32).at[:, :n_exp].set(gW.T)
    neg = jnp.full((1, ep), -1e30, jnp.float32)
    gbp = neg.at[0, :n_exp].set(gb)
    rbp = neg.at[0, :n_exp].set(router_biases)
    idx, w = pl.pallas_call(
        functools.partial(_router_body, n_exp=n_exp),
        grid=(n // tm,),
        in_specs=[
            pl.BlockSpec((tm, emb), lambda i: (i, 0)),
            pl.BlockSpec((emb, ep), lambda i: (0, 0)),
            pl.BlockSpec((1, ep), lambda i: (0, 0)),
            pl.BlockSpec((1, ep), lambda i: (0, 0)),
        ],
        out_specs=[
            pl.BlockSpec((tm, 2), lambda i: (i, 0)),
            pl.BlockSpec((tm, 2), lambda i: (i, 0)),
        ],
        out_shape=[
            jax.ShapeDtypeStruct((n, 2), jnp.int32),
            jax.ShapeDtypeStruct((n, 2), jnp.float32),
        ],
    )(x2d, gwt, gbp, rbp)
    return idx, w


def _sc_gather_rows(src, idx2d):
    m = idx2d.shape[1]
    d = src.shape[1]
    window = 128
    mesh = plsc.VectorSubcoreMesh(core_axis_name="core", subcore_axis_name="subcore")

    @functools.partial(
        pl.kernel,
        out_type=jax.ShapeDtypeStruct((m, d), src.dtype),
        mesh=mesh,
    )
    def gather_kernel(x_hbm, i_hbm, o_hbm):
        def body(i_vmem, o_vmem):
            pltpu.sync_copy(x_hbm.at[i_vmem.at[0]], o_vmem)

        pltpu.emit_pipeline(
            body,
            grid=(m // window,),
            in_specs=[pl.BlockSpec((1, window), lambda i: (0, i))],
            out_specs=[pl.BlockSpec((window, d), lambda i: (i, 0))],
            core_axis_name=("core", "subcore"),
            dimension_semantics=(pltpu.PARALLEL,),
        )(i_hbm, o_hbm)

    return gather_kernel(src, idx2d)


def _sc_scatter_rows(src, idx2d):
    rows, d = src.shape
    m = idx2d.shape[1]
    window = 128
    n_src_blocks = rows // window
    mesh = plsc.VectorSubcoreMesh(core_axis_name="core", subcore_axis_name="subcore")

    @functools.partial(
        pl.kernel,
        out_type=jax.ShapeDtypeStruct((m, d), src.dtype),
        mesh=mesh,
    )
    def scatter_kernel(x_hbm, i_hbm, o_hbm):
        def body(x_vmem, i_vmem):
            pltpu.sync_copy(x_vmem, o_hbm.at[i_vmem.at[0]])

        pltpu.emit_pipeline(
            body,
            grid=(m // window,),
            in_specs=[
                pl.BlockSpec((window, d), lambda i: (i % n_src_blocks, 0)),
                pl.BlockSpec((1, window), lambda i: (0, i)),
            ],
            out_specs=[],
            core_axis_name=("core", "subcore"),
            dimension_semantics=(pltpu.PARALLEL,),
        )(x_hbm, i_hbm)

    return scatter_kernel(src, idx2d)


def _grouped_body(sgid_ref, stile_ref, rs_ref, re_ref,
                  x_ref, w1_ref, wg_ref, w2_ref, o_ref,
                  w1b_ref, wgb_ref, w2b_ref, *, tm):
    s = pl.program_id(0)
    prev_gid = sgid_ref[jnp.maximum(s - 1, 0)]
    new_expert = jnp.logical_or(s == 0, sgid_ref[s] != prev_gid)

    @pl.when(new_expert)
    def _():
        w1b_ref[...] = w1_ref[0].astype(jnp.bfloat16)
        wgb_ref[...] = wg_ref[0].astype(jnp.bfloat16)
        w2b_ref[...] = w2_ref[0].astype(jnp.bfloat16)

    x = x_ref[...]
    a = jnp.dot(x, w1b_ref[...], preferred_element_type=jnp.float32)
    b = jnp.dot(x, wgb_ref[...], preferred_element_type=jnp.float32)
    h = (a * jax.lax.logistic(a) * b).astype(jnp.bfloat16)
    o = jnp.dot(h, w2b_ref[...], preferred_element_type=jnp.float32)
    rows = stile_ref[s] * tm + jax.lax.broadcasted_iota(jnp.int32, (tm, 1), 0)
    keep = (rows >= rs_ref[s]) & (rows < re_ref[s])
    o = jnp.where(keep, o, 0.0).astype(jnp.bfloat16)
    prev_tile = stile_ref[jnp.maximum(s - 1, 0)]
    first_visit = jnp.logical_or(s == 0, stile_ref[s] != prev_tile)

    @pl.when(first_visit)
    def _():
        o_ref[...] = o

    @pl.when(jnp.logical_not(first_visit))
    def _():
        o_ref[...] += o


def _grouped_ffn(xs, W1, Wg, W2, sgid, stile, rs, re, tm, n_steps):
    a_total, emb = xs.shape
    hid = W1.shape[2]
    return pl.pallas_call(
        functools.partial(_grouped_body, tm=tm),
        grid_spec=pltpu.PrefetchScalarGridSpec(
            num_scalar_prefetch=4,
            grid=(n_steps,),
            in_specs=[
                pl.BlockSpec((tm, emb), lambda s, g, t, a, b: (t[s], 0)),
                pl.BlockSpec((1, emb, hid), lambda s, g, t, a, b: (g[s], 0, 0)),
                pl.BlockSpec((1, emb, hid), lambda s, g, t, a, b: (g[s], 0, 0)),
                pl.BlockSpec((1, hid, emb), lambda s, g, t, a, b: (g[s], 0, 0)),
            ],
            out_specs=pl.BlockSpec((tm, emb), lambda s, g, t, a, b: (t[s], 0)),
            scratch_shapes=[
                pltpu.VMEM((emb, hid), jnp.bfloat16),
                pltpu.VMEM((emb, hid), jnp.bfloat16),
                pltpu.VMEM((hid, emb), jnp.bfloat16),
            ],
        ),
        out_shape=jax.ShapeDtypeStruct((a_total, emb), jnp.bfloat16),
    )(sgid, stile, rs, re, xs, W1, Wg, W2)


def _shared_body(x_ref, w1_ref, b1_ref, w2_ref, b2_ref, o_ref,
                 w1b_ref, w2b_ref, *, emb):
    @pl.when(pl.program_id(0) == 0)
    def _():
        w1b_ref[...] = w1_ref[...].astype(jnp.bfloat16)
        w2b_ref[...] = w2_ref[...].astype(jnp.bfloat16)

    x = x_ref[...].astype(jnp.bfloat16)
    h = jnp.dot(x, w1b_ref[...], preferred_element_type=jnp.float32) + b1_ref[...]
    h = (h * jax.lax.logistic(h)).astype(jnp.bfloat16)
    o = jnp.dot(h, w2b_ref[...], preferred_element_type=jnp.float32) + b2_ref[...]
    o_ref[...] = o


def _shared_ffn(x2d, sW1, sb1, sW2, sb2):
    n, emb = x2d.shape
    hid = sW1.shape[2]
    h2 = 2 * hid
    tm = 256
    w1c = jnp.concatenate([sW1[0], sW1[1]], axis=1)
    b1c = sb1.reshape(1, h2)
    w2c = sW2.reshape(h2, emb)
    b2c = (sb2[0] + sb2[1]).reshape(1, emb)
    return pl.pallas_call(
        functools.partial(_shared_body, emb=emb),
        grid=(n // tm,),
        in_specs=[
            pl.BlockSpec((tm, emb), lambda i: (i, 0)),
            pl.BlockSpec((emb, h2), lambda i: (0, 0)),
            pl.BlockSpec((1, h2), lambda i: (0, 0)),
            pl.BlockSpec((h2, emb), lambda i: (0, 0)),
            pl.BlockSpec((1, emb), lambda i: (0, 0)),
        ],
        out_specs=pl.BlockSpec((tm, emb), lambda i: (i, 0)),
        out_shape=jax.ShapeDtypeStruct((n, emb), jnp.float32),
        scratch_shapes=[
            pltpu.VMEM((emb, h2), jnp.bfloat16),
            pltpu.VMEM((h2, emb), jnp.bfloat16),
        ],
    )(x2d, w1c, b1c, w2c, b2c)


def _combine_body(s_ref, g_ref, w_ref, o_ref, *, emb):
    g = g_ref[...].astype(jnp.float32)
    w = w_ref[...]
    o_ref[...] = (s_ref[...] + w[:, 0:1] * g[:, :emb]
                  + w[:, 1:2] * g[:, emb:])


def _combine(shared_out, g2, w):
    n, emb = shared_out.shape
    tm = 512
    return pl.pallas_call(
        functools.partial(_combine_body, emb=emb),
        grid=(n // tm,),
        in_specs=[
            pl.BlockSpec((tm, emb), lambda i: (i, 0)),
            pl.BlockSpec((tm, 2 * emb), lambda i: (i, 0)),
            pl.BlockSpec((tm, 2), lambda i: (i, 0)),
        ],
        out_specs=pl.BlockSpec((tm, emb), lambda i: (i, 0)),
        out_shape=jax.ShapeDtypeStruct((n, emb), jnp.float32),
    )(shared_out, g2, w)


def _metadata_body(idx_ref, pos_ref, sgid_ref, stile_ref, rs_ref, re_ref,
                   *, n_exp, tm, n_steps_pad, n_tok):
    f32 = jnp.float32
    idx = idx_ref[...]
    e_iota = jax.lax.broadcasted_iota(jnp.int32, (n_tok, n_exp), 1)
    oh0 = (idx[:, 0:1] == e_iota).astype(f32)
    oh1 = (idx[:, 1:2] == e_iota).astype(f32)

    ohb = jnp.concatenate([oh0, oh1], axis=1)
    acc = ohb
    sh = 1
    while sh < n_tok:
        acc = acc + jnp.concatenate(
            [jnp.zeros((sh, 2 * n_exp), f32), acc[: n_tok - sh, :]], axis=0)
        sh *= 2
    tot = acc[n_tok - 1:n_tok, :]
    tot0 = tot[:, :n_exp]
    tot1 = tot[:, n_exp:]
    ex0 = acc[:, :n_exp] - oh0
    ex1 = acc[:, n_exp:] - oh1
    counts = tot0 + tot1

    def _lane_prefix_excl(v):
        acc = v
        sh = 1
        while sh < n_exp:
            acc = acc + jnp.concatenate(
                [jnp.zeros((1, sh), f32), acc[:, : n_exp - sh]], axis=1)
            sh *= 2
        return acc - v

    offs = _lane_prefix_excl(counts)
    pos0 = jnp.sum(jnp.where(oh0 > 0.5, ex0 + offs, 0.0),
                   axis=1, keepdims=True)
    pos1 = jnp.sum(jnp.where(oh1 > 0.5, ex1 + offs + tot0, 0.0),
                   axis=1, keepdims=True)
    pos_ref[...] = jnp.round(
        jnp.concatenate([pos0, pos1], axis=1)).astype(jnp.int32)

    ends = offs + counts
    tfirst = jnp.floor(offs / tm)
    tlast = jnp.where(counts > 0, jnp.floor((ends - 1) / tm), tfirst)
    steps_g = jnp.where(counts > 0, tlast - tfirst + 1, 0.0)
    cs = _lane_prefix_excl(steps_g) + steps_g
    total = jnp.sum(steps_g)
    s_iota = jax.lax.broadcasted_iota(
        jnp.int32, (n_steps_pad, 1), 0).astype(f32)
    cs_b = jnp.broadcast_to(cs, (n_steps_pad, n_exp))
    sgid = jnp.sum((cs_b <= s_iota).astype(f32), axis=1, keepdims=True)
    sgid = jnp.minimum(sgid, float(n_exp - 1))
    onehot_sg = (sgid == jax.lax.broadcasted_iota(
        jnp.int32, (n_steps_pad, n_exp), 1).astype(f32))
    onehot_sg = onehot_sg.astype(f32)

    def gath(v):
        return jnp.sum(onehot_sg * v, axis=1, keepdims=True)

    within = s_iota - (gath(cs) - gath(steps_g))
    stile = gath(tfirst) + within
    valid = s_iota < total
    n_tiles = float(2 * n_tok // tm)
    stile = jnp.where(valid, stile, n_tiles - 1)
    rs = jnp.where(valid, jnp.maximum(gath(offs), stile * tm), 0.0)
    re = jnp.where(valid, jnp.minimum(gath(ends), (stile + 1) * tm), 0.0)
    sgid_ref[...] = sgid.astype(jnp.int32)
    stile_ref[...] = stile.astype(jnp.int32)
    rs_ref[...] = rs.astype(jnp.int32)
    re_ref[...] = re.astype(jnp.int32)


def _dispatch_metadata(idx, n_exp, tm):
    n_tok = idx.shape[0]
    n_tiles = 2 * n_tok // tm
    n_steps = n_tiles + n_exp - 1
    n_steps_pad = ((n_steps + 7) // 8) * 8
    pos01, sgid, stile, rs, re = pl.pallas_call(
        functools.partial(_metadata_body, n_exp=n_exp, tm=tm,
                          n_steps_pad=n_steps_pad, n_tok=n_tok),
        grid=(1,),
        in_specs=[pl.BlockSpec((n_tok, 2), lambda i: (0, 0))],
        out_specs=[
            pl.BlockSpec((n_tok, 2), lambda i: (0, 0)),
            pl.BlockSpec((n_steps_pad, 1), lambda i: (0, 0)),
            pl.BlockSpec((n_steps_pad, 1), lambda i: (0, 0)),
            pl.BlockSpec((n_steps_pad, 1), lambda i: (0, 0)),
            pl.BlockSpec((n_steps_pad, 1), lambda i: (0, 0)),
        ],
        out_shape=[
            jax.ShapeDtypeStruct((n_tok, 2), jnp.int32),
            jax.ShapeDtypeStruct((n_steps_pad, 1), jnp.int32),
            jax.ShapeDtypeStruct((n_steps_pad, 1), jnp.int32),
            jax.ShapeDtypeStruct((n_steps_pad, 1), jnp.int32),
            jax.ShapeDtypeStruct((n_steps_pad, 1), jnp.int32),
        ],
    )(idx)
    return (pos01, sgid.reshape(-1), stile.reshape(-1), rs.reshape(-1),
            re.reshape(-1), n_steps_pad)


def kernel(x, W1, Wg, W2, sW1, sb1, sW2, sb2, gW, gb, router_biases):
    b, s_, emb = x.shape
    x2d = x.reshape(-1, emb)
    n = x2d.shape[0]
    n_exp = W1.shape[0]
    tm = 128

    idx, w = _router(x2d, gW, gb, router_biases)
    pos01, sgid, stile, rs, re, n_steps = _dispatch_metadata(idx, n_exp, tm)

    third = jnp.arange(3, dtype=jnp.int32)
    s_idx = (pos01.T.reshape(-1, 1) * 3 + third).reshape(1, -1)
    g_idx = (pos01.reshape(-1, 1) * 3 + third).reshape(1, -1)

    def _pack(a):
        return jax.lax.bitcast_convert_type(
            a.reshape(a.shape[0], a.shape[1] // 2, 2), jnp.int32)

    def _unpack(a):
        return jax.lax.bitcast_convert_type(a, jnp.bfloat16).reshape(
            a.shape[0], a.shape[1] * 2)

    x2dp = _pack(x2d.astype(jnp.bfloat16))
    xs = _sc_scatter_rows(x2dp.reshape(3 * n, emb // 6), s_idx)
    ys = _grouped_ffn(_unpack(xs.reshape(2 * n, emb // 2)), W1, Wg, W2,
                      sgid, stile, rs, re, tm, n_steps)
    ysp = _pack(ys)
    g = _sc_gather_rows(ysp.reshape(6 * n, emb // 6), g_idx)
    g2 = _unpack(g.reshape(n, emb))
    shared_out = _shared_ffn(x2d, sW1, sb1, sW2, sb2)
    out = _combine(shared_out, g2, w)
    return out.reshape(b, s_, emb)


# speedup vs baseline: 2.0030x; 2.0030x over previous
import functools

import jax
import jax.numpy as jnp
from jax.experimental import pallas as pl
from jax.experimental.pallas import tpu as pltpu
from jax.experimental.pallas import tpu_sc as plsc


def _router_body(x_ref, gwt_ref, gb_ref, rb_ref, idx_ref, w_ref, *, n_exp):
    x = x_ref[...]
    logits = jnp.dot(x, gwt_ref[...], preferred_element_type=jnp.float32)
    logits = logits + gb_ref[...]
    m = jnp.max(logits, axis=-1, keepdims=True)
    p = jnp.exp(logits - m)
    probas = p / jnp.sum(p, axis=-1, keepdims=True)
    biased = probas + rb_ref[...]
    col = jax.lax.broadcasted_iota(jnp.int32, biased.shape, 1)
    i1 = jnp.argmax(biased, axis=-1, keepdims=True).astype(jnp.int32)
    biased2 = jnp.where(col == i1, -1e30, biased)
    i2 = jnp.argmax(biased2, axis=-1, keepdims=True).astype(jnp.int32)
    p1 = jnp.sum(jnp.where(col == i1, probas, 0.0), axis=-1, keepdims=True)
    p2 = jnp.sum(jnp.where(col == i2, probas, 0.0), axis=-1, keepdims=True)
    tot = p1 + p2
    idx_ref[...] = jnp.concatenate([i1, i2], axis=1)
    w_ref[...] = jnp.concatenate([p1 / tot, p2 / tot], axis=1)


def _router(x2d, gW, gb, router_biases):
    n, emb = x2d.shape
    n_exp = gW.shape[0]
    ep = 128
    tm = 512
    gwt = jnp.zeros((emb, ep), jnp.float32).at[:, :n_exp].set(gW.T)
    neg = jnp.full((1, ep), -1e30, jnp.float32)
    gbp = neg.at[0, :n_exp].set(gb)
    rbp = neg.at[0, :n_exp].set(router_biases)
    idx, w = pl.pallas_call(
        functools.partial(_router_body, n_exp=n_exp),
        grid=(n // tm,),
        in_specs=[
            pl.BlockSpec((tm, emb), lambda i: (i, 0)),
            pl.BlockSpec((emb, ep), lambda i: (0, 0)),
            pl.BlockSpec((1, ep), lambda i: (0, 0)),
            pl.BlockSpec((1, ep), lambda i: (0, 0)),
        ],
        out_specs=[
            pl.BlockSpec((tm, 2), lambda i: (i, 0)),
            pl.BlockSpec((tm, 2), lambda i: (i, 0)),
        ],
        out_shape=[
            jax.ShapeDtypeStruct((n, 2), jnp.int32),
            jax.ShapeDtypeStruct((n, 2), jnp.float32),
        ],
    )(x2d, gwt, gbp, rbp)
    return idx, w


def _sc_gather_rows(src, idx2d):
    m = idx2d.shape[1]
    d = src.shape[1]
    window = 128
    mesh = plsc.VectorSubcoreMesh(core_axis_name="core", subcore_axis_name="subcore")

    @functools.partial(
        pl.kernel,
        out_type=jax.ShapeDtypeStruct((m, d), src.dtype),
        mesh=mesh,
    )
    def gather_kernel(x_hbm, i_hbm, o_hbm):
        def body(i_vmem, o_vmem):
            pltpu.sync_copy(x_hbm.at[i_vmem.at[0]], o_vmem)

        pltpu.emit_pipeline(
            body,
            grid=(m // window,),
            in_specs=[pl.BlockSpec((1, window), lambda i: (0, i))],
            out_specs=[pl.BlockSpec((window, d), lambda i: (i, 0))],
            core_axis_name=("core", "subcore"),
            dimension_semantics=(pltpu.PARALLEL,),
        )(i_hbm, o_hbm)

    return gather_kernel(src, idx2d)


def _sc_scatter_rows(src, idx2d):
    rows, d = src.shape
    m = idx2d.shape[1]
    window = 128
    n_src_blocks = rows // window
    mesh = plsc.VectorSubcoreMesh(core_axis_name="core", subcore_axis_name="subcore")

    @functools.partial(
        pl.kernel,
        out_type=jax.ShapeDtypeStruct((m, d), src.dtype),
        mesh=mesh,
    )
    def scatter_kernel(x_hbm, i_hbm, o_hbm):
        def body(x_vmem, i_vmem):
            pltpu.sync_copy(x_vmem, o_hbm.at[i_vmem.at[0]])

        pltpu.emit_pipeline(
            body,
            grid=(m // window,),
            in_specs=[
                pl.BlockSpec((window, d), lambda i: (i % n_src_blocks, 0)),
                pl.BlockSpec((1, window), lambda i: (0, i)),
            ],
            out_specs=[],
            core_axis_name=("core", "subcore"),
            dimension_semantics=(pltpu.PARALLEL,),
        )(x_hbm, i_hbm)

    return scatter_kernel(src, idx2d)


def _grouped_body(sgid_ref, stile_ref, rs_ref, re_ref,
                  x_ref, w1_ref, wg_ref, w2_ref, o_ref,
                  w1b_ref, wgb_ref, w2b_ref, *, tm):
    s = pl.program_id(0)
    prev_gid = sgid_ref[jnp.maximum(s - 1, 0)]
    new_expert = jnp.logical_or(s == 0, sgid_ref[s] != prev_gid)

    @pl.when(new_expert)
    def _():
        w1b_ref[...] = w1_ref[0].astype(jnp.bfloat16)
        wgb_ref[...] = wg_ref[0].astype(jnp.bfloat16)
        w2b_ref[...] = w2_ref[0].astype(jnp.bfloat16)

    x = x_ref[...].astype(jnp.bfloat16)
    a = jnp.dot(x, w1b_ref[...], preferred_element_type=jnp.float32)
    b = jnp.dot(x, wgb_ref[...], preferred_element_type=jnp.float32)
    h = (a * jax.lax.logistic(a) * b).astype(jnp.bfloat16)
    o = jnp.dot(h, w2b_ref[...], preferred_element_type=jnp.float32)
    rows = stile_ref[s] * tm + jax.lax.broadcasted_iota(jnp.int32, (tm, 1), 0)
    keep = (rows >= rs_ref[s]) & (rows < re_ref[s])
    o = jnp.where(keep, o, 0.0)
    prev_tile = stile_ref[jnp.maximum(s - 1, 0)]
    first_visit = jnp.logical_or(s == 0, stile_ref[s] != prev_tile)

    @pl.when(first_visit)
    def _():
        o_ref[...] = o

    @pl.when(jnp.logical_not(first_visit))
    def _():
        o_ref[...] += o


def _grouped_ffn(xs, W1, Wg, W2, sgid, stile, rs, re, tm, n_steps):
    a_total, emb = xs.shape
    hid = W1.shape[2]
    return pl.pallas_call(
        functools.partial(_grouped_body, tm=tm),
        grid_spec=pltpu.PrefetchScalarGridSpec(
            num_scalar_prefetch=4,
            grid=(n_steps,),
            in_specs=[
                pl.BlockSpec((tm, emb), lambda s, g, t, a, b: (t[s], 0)),
                pl.BlockSpec((1, emb, hid), lambda s, g, t, a, b: (g[s], 0, 0)),
                pl.BlockSpec((1, emb, hid), lambda s, g, t, a, b: (g[s], 0, 0)),
                pl.BlockSpec((1, hid, emb), lambda s, g, t, a, b: (g[s], 0, 0)),
            ],
            out_specs=pl.BlockSpec((tm, emb), lambda s, g, t, a, b: (t[s], 0)),
            scratch_shapes=[
                pltpu.VMEM((emb, hid), jnp.bfloat16),
                pltpu.VMEM((emb, hid), jnp.bfloat16),
                pltpu.VMEM((hid, emb), jnp.bfloat16),
            ],
        ),
        out_shape=jax.ShapeDtypeStruct((a_total, emb), jnp.float32),
    )(sgid, stile, rs, re, xs, W1, Wg, W2)


def _shared_body(x_ref, w1_ref, b1_ref, w2_ref, b2_ref, g_ref, w_ref, o_ref,
                 w1b_ref, w2b_ref, *, emb):
    @pl.when(pl.program_id(0) == 0)
    def _():
        w1b_ref[...] = w1_ref[...].astype(jnp.bfloat16)
        w2b_ref[...] = w2_ref[...].astype(jnp.bfloat16)

    x = x_ref[...].astype(jnp.bfloat16)
    h = jnp.dot(x, w1b_ref[...], preferred_element_type=jnp.float32) + b1_ref[...]
    h = (h * jax.lax.logistic(h)).astype(jnp.bfloat16)
    o = jnp.dot(h, w2b_ref[...], preferred_element_type=jnp.float32) + b2_ref[...]
    g = g_ref[...]
    w = w_ref[...]
    o_ref[...] = o + w[:, 0:1] * g[:, :emb] + w[:, 1:2] * g[:, emb:]


def _shared_combine(x2d, sW1, sb1, sW2, sb2, g2, w):
    n, emb = x2d.shape
    hid = sW1.shape[2]
    h2 = 2 * hid
    tm = 256
    w1c = jnp.concatenate([sW1[0], sW1[1]], axis=1)
    b1c = sb1.reshape(1, h2)
    w2c = sW2.reshape(h2, emb)
    b2c = (sb2[0] + sb2[1]).reshape(1, emb)
    return pl.pallas_call(
        functools.partial(_shared_body, emb=emb),
        grid=(n // tm,),
        in_specs=[
            pl.BlockSpec((tm, emb), lambda i: (i, 0)),
            pl.BlockSpec((emb, h2), lambda i: (0, 0)),
            pl.BlockSpec((1, h2), lambda i: (0, 0)),
            pl.BlockSpec((h2, emb), lambda i: (0, 0)),
            pl.BlockSpec((1, emb), lambda i: (0, 0)),
            pl.BlockSpec((tm, 2 * emb), lambda i: (i, 0)),
            pl.BlockSpec((tm, 2), lambda i: (i, 0)),
        ],
        out_specs=pl.BlockSpec((tm, emb), lambda i: (i, 0)),
        out_shape=jax.ShapeDtypeStruct((n, emb), jnp.float32),
        scratch_shapes=[
            pltpu.VMEM((emb, h2), jnp.bfloat16),
            pltpu.VMEM((h2, emb), jnp.bfloat16),
        ],
    )(x2d, w1c, b1c, w2c, b2c, g2, w)


def _metadata_body(idx_ref, pos_ref, sgid_ref, stile_ref, rs_ref, re_ref,
                   *, n_exp, tm, n_steps_pad, n_tok):
    f32 = jnp.float32
    idx = idx_ref[...]
    e_iota = jax.lax.broadcasted_iota(jnp.int32, (n_tok, n_exp), 1)
    oh0 = (idx[:, 0:1] == e_iota).astype(f32)
    oh1 = (idx[:, 1:2] == e_iota).astype(f32)

    ohb = jnp.concatenate([oh0, oh1], axis=1)
    acc = ohb
    sh = 1
    while sh < n_tok:
        acc = acc + jnp.concatenate(
            [jnp.zeros((sh, 2 * n_exp), f32), acc[: n_tok - sh, :]], axis=0)
        sh *= 2
    tot = acc[n_tok - 1:n_tok, :]
    tot0 = tot[:, :n_exp]
    tot1 = tot[:, n_exp:]
    ex0 = acc[:, :n_exp] - oh0
    ex1 = acc[:, n_exp:] - oh1
    counts = tot0 + tot1

    def _lane_prefix_excl(v):
        acc = v
        sh = 1
        while sh < n_exp:
            acc = acc + jnp.concatenate(
                [jnp.zeros((1, sh), f32), acc[:, : n_exp - sh]], axis=1)
            sh *= 2
        return acc - v

    offs = _lane_prefix_excl(counts)
    pos0 = jnp.sum(jnp.where(oh0 > 0.5, ex0 + offs, 0.0),
                   axis=1, keepdims=True)
    pos1 = jnp.sum(jnp.where(oh1 > 0.5, ex1 + offs + tot0, 0.0),
                   axis=1, keepdims=True)
    pos_ref[...] = jnp.round(
        jnp.concatenate([pos0, pos1], axis=1)).astype(jnp.int32)

    ends = offs + counts
    tfirst = jnp.floor(offs / tm)
    tlast = jnp.where(counts > 0, jnp.floor((ends - 1) / tm), tfirst)
    steps_g = jnp.where(counts > 0, tlast - tfirst + 1, 0.0)
    cs = _lane_prefix_excl(steps_g) + steps_g
    total = jnp.sum(steps_g)
    s_iota = jax.lax.broadcasted_iota(
        jnp.int32, (n_steps_pad, 1), 0).astype(f32)
    cs_b = jnp.broadcast_to(cs, (n_steps_pad, n_exp))
    sgid = jnp.sum((cs_b <= s_iota).astype(f32), axis=1, keepdims=True)
    sgid = jnp.minimum(sgid, float(n_exp - 1))
    onehot_sg = (sgid == jax.lax.broadcasted_iota(
        jnp.int32, (n_steps_pad, n_exp), 1).astype(f32))
    onehot_sg = onehot_sg.astype(f32)

    def gath(v):
        return jnp.sum(onehot_sg * v, axis=1, keepdims=True)

    within = s_iota - (gath(cs) - gath(steps_g))
    stile = gath(tfirst) + within
    valid = s_iota < total
    n_tiles = float(2 * n_tok // tm)
    stile = jnp.where(valid, stile, n_tiles - 1)
    rs = jnp.where(valid, jnp.maximum(gath(offs), stile * tm), 0.0)
    re = jnp.where(valid, jnp.minimum(gath(ends), (stile + 1) * tm), 0.0)
    sgid_ref[...] = sgid.astype(jnp.int32)
    stile_ref[...] = stile.astype(jnp.int32)
    rs_ref[...] = rs.astype(jnp.int32)
    re_ref[...] = re.astype(jnp.int32)


def _dispatch_metadata(idx, n_exp, tm):
    n_tok = idx.shape[0]
    n_tiles = 2 * n_tok // tm
    n_steps = n_tiles + n_exp - 1
    n_steps_pad = ((n_steps + 7) // 8) * 8
    pos01, sgid, stile, rs, re = pl.pallas_call(
        functools.partial(_metadata_body, n_exp=n_exp, tm=tm,
                          n_steps_pad=n_steps_pad, n_tok=n_tok),
        grid=(1,),
        in_specs=[pl.BlockSpec((n_tok, 2), lambda i: (0, 0))],
        out_specs=[
            pl.BlockSpec((n_tok, 2), lambda i: (0, 0)),
            pl.BlockSpec((n_steps_pad, 1), lambda i: (0, 0)),
            pl.BlockSpec((n_steps_pad, 1), lambda i: (0, 0)),
            pl.BlockSpec((n_steps_pad, 1), lambda i: (0, 0)),
            pl.BlockSpec((n_steps_pad, 1), lambda i: (0, 0)),
        ],
        out_shape=[
            jax.ShapeDtypeStruct((n_tok, 2), jnp.int32),
            jax.ShapeDtypeStruct((n_steps_pad, 1), jnp.int32),
            jax.ShapeDtypeStruct((n_steps_pad, 1), jnp.int32),
            jax.ShapeDtypeStruct((n_steps_pad, 1), jnp.int32),
            jax.ShapeDtypeStruct((n_steps_pad, 1), jnp.int32),
        ],
    )(idx)
    return (pos01, sgid.reshape(-1), stile.reshape(-1), rs.reshape(-1),
            re.reshape(-1), n_steps_pad)


def kernel(x, W1, Wg, W2, sW1, sb1, sW2, sb2, gW, gb, router_biases):
    b, s_, emb = x.shape
    x2d = x.reshape(-1, emb)
    n = x2d.shape[0]
    n_exp = W1.shape[0]
    tm = 128

    idx, w = _router(x2d, gW, gb, router_biases)
    pos01, sgid, stile, rs, re, n_steps = _dispatch_metadata(idx, n_exp, tm)

    half = jnp.arange(2, dtype=jnp.int32)
    s_idx = (pos01.T.reshape(-1, 1) * 2 + half).reshape(1, -1)
    g_idx = (pos01.reshape(-1, 1) * 2 + half).reshape(1, -1)

    xs = _sc_scatter_rows(x2d.reshape(2 * n, emb // 2), s_idx)
    ys = _grouped_ffn(xs.reshape(2 * n, emb), W1, Wg, W2,
                      sgid, stile, rs, re, tm, n_steps)
    g = _sc_gather_rows(ys.reshape(4 * n, emb // 2), g_idx)
    g2 = g.reshape(n, 2 * emb)
    out = _shared_combine(x2d, sW1, sb1, sW2, sb2, g2, w)
    return out.reshape(b, s_, emb)


# speedup vs baseline: 2.0822x; 1.0395x over previous
import functools

import jax
import jax.numpy as jnp
from jax.experimental import pallas as pl
from jax.experimental.pallas import tpu as pltpu
from jax.experimental.pallas import tpu_sc as plsc


def _router_body(x_ref, gwt_ref, gb_ref, rb_ref, idx_ref, w_ref, *, n_exp):
    x = x_ref[...]
    logits = jnp.dot(x, gwt_ref[...], preferred_element_type=jnp.float32)
    logits = logits + gb_ref[...]
    m = jnp.max(logits, axis=-1, keepdims=True)
    p = jnp.exp(logits - m)
    probas = p / jnp.sum(p, axis=-1, keepdims=True)
    biased = probas + rb_ref[...]
    col = jax.lax.broadcasted_iota(jnp.int32, biased.shape, 1)
    i1 = jnp.argmax(biased, axis=-1, keepdims=True).astype(jnp.int32)
    biased2 = jnp.where(col == i1, -1e30, biased)
    i2 = jnp.argmax(biased2, axis=-1, keepdims=True).astype(jnp.int32)
    p1 = jnp.sum(jnp.where(col == i1, probas, 0.0), axis=-1, keepdims=True)
    p2 = jnp.sum(jnp.where(col == i2, probas, 0.0), axis=-1, keepdims=True)
    tot = p1 + p2
    idx_ref[...] = jnp.concatenate([i1, i2], axis=1)
    w_ref[...] = jnp.concatenate([p1 / tot, p2 / tot], axis=1)


def _router(x2d, gW, gb, router_biases):
    n, emb = x2d.shape
    n_exp = gW.shape[0]
    ep = 128
    tm = 512
    gwt = jnp.zeros((emb, ep), jnp.float32).at[:, :n_exp].set(gW.T)
    neg = jnp.full((1, ep), -1e30, jnp.float32)
    gbp = neg.at[0, :n_exp].set(gb)
    rbp = neg.at[0, :n_exp].set(router_biases)
    idx, w = pl.pallas_call(
        functools.partial(_router_body, n_exp=n_exp),
        grid=(n // tm,),
        in_specs=[
            pl.BlockSpec((tm, emb), lambda i: (i, 0)),
            pl.BlockSpec((emb, ep), lambda i: (0, 0)),
            pl.BlockSpec((1, ep), lambda i: (0, 0)),
            pl.BlockSpec((1, ep), lambda i: (0, 0)),
        ],
        out_specs=[
            pl.BlockSpec((tm, 2), lambda i: (i, 0)),
            pl.BlockSpec((tm, 2), lambda i: (i, 0)),
        ],
        out_shape=[
            jax.ShapeDtypeStruct((n, 2), jnp.int32),
            jax.ShapeDtypeStruct((n, 2), jnp.float32),
        ],
    )(x2d, gwt, gbp, rbp)
    return idx, w


def _sc_gather_rows(src, idx2d):
    m = idx2d.shape[1]
    d = src.shape[1]
    window = 128
    mesh = plsc.VectorSubcoreMesh(core_axis_name="core", subcore_axis_name="subcore")

    @functools.partial(
        pl.kernel,
        out_type=jax.ShapeDtypeStruct((m, d), src.dtype),
        mesh=mesh,
    )
    def gather_kernel(x_hbm, i_hbm, o_hbm):
        def body(i_vmem, o_vmem):
            pltpu.sync_copy(x_hbm.at[i_vmem.at[0]], o_vmem)

        pltpu.emit_pipeline(
            body,
            grid=(m // window,),
            in_specs=[pl.BlockSpec((1, window), lambda i: (0, i))],
            out_specs=[pl.BlockSpec((window, d), lambda i: (i, 0))],
            core_axis_name=("core", "subcore"),
            dimension_semantics=(pltpu.PARALLEL,),
        )(i_hbm, o_hbm)

    return gather_kernel(src, idx2d)


def _sc_scatter_rows(src, idx2d):
    rows, d = src.shape
    m = idx2d.shape[1]
    window = 128
    n_src_blocks = rows // window
    mesh = plsc.VectorSubcoreMesh(core_axis_name="core", subcore_axis_name="subcore")

    @functools.partial(
        pl.kernel,
        out_type=jax.ShapeDtypeStruct((m, d), src.dtype),
        mesh=mesh,
    )
    def scatter_kernel(x_hbm, i_hbm, o_hbm):
        def body(x_vmem, i_vmem):
            pltpu.sync_copy(x_vmem, o_hbm.at[i_vmem.at[0]])

        pltpu.emit_pipeline(
            body,
            grid=(m // window,),
            in_specs=[
                pl.BlockSpec((window, d), lambda i: (i % n_src_blocks, 0)),
                pl.BlockSpec((1, window), lambda i: (0, i)),
            ],
            out_specs=[],
            core_axis_name=("core", "subcore"),
            dimension_semantics=(pltpu.PARALLEL,),
        )(x_hbm, i_hbm)

    return scatter_kernel(src, idx2d)


def _grouped_body(sgid_ref, stile_ref, rs_ref, re_ref,
                  x_ref, w1_ref, wg_ref, w2_ref, o_ref,
                  w1b_ref, wgb_ref, w2b_ref, *, tm):
    s = pl.program_id(0)
    prev_gid = sgid_ref[jnp.maximum(s - 1, 0)]
    new_expert = jnp.logical_or(s == 0, sgid_ref[s] != prev_gid)

    @pl.when(new_expert)
    def _():
        w1b_ref[...] = w1_ref[0].astype(jnp.bfloat16)
        wgb_ref[...] = wg_ref[0].astype(jnp.bfloat16)
        w2b_ref[...] = w2_ref[0].astype(jnp.bfloat16)

    x = x_ref[...].astype(jnp.bfloat16)
    a = jnp.dot(x, w1b_ref[...], preferred_element_type=jnp.float32)
    b = jnp.dot(x, wgb_ref[...], preferred_element_type=jnp.float32)
    h = (a * jax.lax.logistic(a) * b).astype(jnp.bfloat16)
    o = jnp.dot(h, w2b_ref[...], preferred_element_type=jnp.float32)
    rows = stile_ref[s] * tm + jax.lax.broadcasted_iota(jnp.int32, (tm, 1), 0)
    keep = (rows >= rs_ref[s]) & (rows < re_ref[s])
    o = jnp.where(keep, o, 0.0)
    prev_tile = stile_ref[jnp.maximum(s - 1, 0)]
    first_visit = jnp.logical_or(s == 0, stile_ref[s] != prev_tile)

    @pl.when(first_visit)
    def _():
        o_ref[...] = o

    @pl.when(jnp.logical_not(first_visit))
    def _():
        o_ref[...] += o


def _grouped_ffn(xs, W1, Wg, W2, sgid, stile, rs, re, tm, n_steps):
    a_total, emb = xs.shape
    hid = W1.shape[2]
    return pl.pallas_call(
        functools.partial(_grouped_body, tm=tm),
        grid_spec=pltpu.PrefetchScalarGridSpec(
            num_scalar_prefetch=4,
            grid=(n_steps,),
            in_specs=[
                pl.BlockSpec((tm, emb), lambda s, g, t, a, b: (t[s], 0)),
                pl.BlockSpec((1, emb, hid), lambda s, g, t, a, b: (g[s], 0, 0)),
                pl.BlockSpec((1, emb, hid), lambda s, g, t, a, b: (g[s], 0, 0)),
                pl.BlockSpec((1, hid, emb), lambda s, g, t, a, b: (g[s], 0, 0)),
            ],
            out_specs=pl.BlockSpec((tm, emb), lambda s, g, t, a, b: (t[s], 0)),
            scratch_shapes=[
                pltpu.VMEM((emb, hid), jnp.bfloat16),
                pltpu.VMEM((emb, hid), jnp.bfloat16),
                pltpu.VMEM((hid, emb), jnp.bfloat16),
            ],
        ),
        out_shape=jax.ShapeDtypeStruct((a_total, emb), jnp.float32),
    )(sgid, stile, rs, re, xs, W1, Wg, W2)


def _shared_body(x_ref, w1_ref, b1_ref, w2_ref, b2_ref, g_ref, w_ref, o_ref,
                 w1b_ref, w2b_ref, *, emb):
    @pl.when(pl.program_id(0) == 0)
    def _():
        w1b_ref[...] = w1_ref[...].astype(jnp.bfloat16)
        w2b_ref[...] = w2_ref[...].astype(jnp.bfloat16)

    x = x_ref[...].astype(jnp.bfloat16)
    h = jnp.dot(x, w1b_ref[...], preferred_element_type=jnp.float32) + b1_ref[...]
    h = (h * jax.lax.logistic(h)).astype(jnp.bfloat16)
    o = jnp.dot(h, w2b_ref[...], preferred_element_type=jnp.float32) + b2_ref[...]
    g = g_ref[...]
    w = w_ref[...]
    o_ref[...] = o + w[:, 0:1] * g[:, :emb] + w[:, 1:2] * g[:, emb:]


def _shared_combine(x2d, sW1, sb1, sW2, sb2, g2, w):
    n, emb = x2d.shape
    hid = sW1.shape[2]
    h2 = 2 * hid
    tm = 256
    w1c = jnp.concatenate([sW1[0], sW1[1]], axis=1)
    b1c = sb1.reshape(1, h2)
    w2c = sW2.reshape(h2, emb)
    b2c = (sb2[0] + sb2[1]).reshape(1, emb)
    return pl.pallas_call(
        functools.partial(_shared_body, emb=emb),
        grid=(n // tm,),
        in_specs=[
            pl.BlockSpec((tm, emb), lambda i: (i, 0)),
            pl.BlockSpec((emb, h2), lambda i: (0, 0)),
            pl.BlockSpec((1, h2), lambda i: (0, 0)),
            pl.BlockSpec((h2, emb), lambda i: (0, 0)),
            pl.BlockSpec((1, emb), lambda i: (0, 0)),
            pl.BlockSpec((tm, 2 * emb), lambda i: (i, 0)),
            pl.BlockSpec((tm, 2), lambda i: (i, 0)),
        ],
        out_specs=pl.BlockSpec((tm, emb), lambda i: (i, 0)),
        out_shape=jax.ShapeDtypeStruct((n, emb), jnp.float32),
        scratch_shapes=[
            pltpu.VMEM((emb, h2), jnp.bfloat16),
            pltpu.VMEM((h2, emb), jnp.bfloat16),
        ],
    )(x2d, w1c, b1c, w2c, b2c, g2, w)


def _metadata_body(idx_ref, pos_ref, sgid_ref, stile_ref, rs_ref, re_ref,
                   *, n_exp, tm, n_steps_pad, n_tok):
    f32 = jnp.float32
    idx = idx_ref[...]
    e_iota = jax.lax.broadcasted_iota(jnp.int32, (n_tok, n_exp), 1)
    oh0 = (idx[:, 0:1] == e_iota).astype(f32)
    oh1 = (idx[:, 1:2] == e_iota).astype(f32)

    ohb = jnp.concatenate([oh0, oh1], axis=1)
    acc = ohb
    sh = 1
    while sh < n_tok:
        acc = acc + jnp.concatenate(
            [jnp.zeros((sh, 2 * n_exp), f32), acc[: n_tok - sh, :]], axis=0)
        sh *= 2
    tot = acc[n_tok - 1:n_tok, :]
    tot0 = tot[:, :n_exp]
    tot1 = tot[:, n_exp:]
    ex0 = acc[:, :n_exp] - oh0
    ex1 = acc[:, n_exp:] - oh1
    counts = tot0 + tot1

    def _lane_prefix_excl(v):
        acc = v
        sh = 1
        while sh < n_exp:
            acc = acc + jnp.concatenate(
                [jnp.zeros((1, sh), f32), acc[:, : n_exp - sh]], axis=1)
            sh *= 2
        return acc - v

    offs = _lane_prefix_excl(counts)
    pos0 = jnp.sum(jnp.where(oh0 > 0.5, ex0 + offs, 0.0),
                   axis=1, keepdims=True)
    pos1 = jnp.sum(jnp.where(oh1 > 0.5, ex1 + offs + tot0, 0.0),
                   axis=1, keepdims=True)
    pos_ref[...] = jnp.round(
        jnp.concatenate([pos0, pos1], axis=1)).astype(jnp.int32)

    ends = offs + counts
    tfirst = jnp.floor(offs / tm)
    tlast = jnp.where(counts > 0, jnp.floor((ends - 1) / tm), tfirst)
    steps_g = jnp.where(counts > 0, tlast - tfirst + 1, 0.0)
    cs = _lane_prefix_excl(steps_g) + steps_g
    total = jnp.sum(steps_g)
    s_iota = jax.lax.broadcasted_iota(
        jnp.int32, (n_steps_pad, 1), 0).astype(f32)
    cs_b = jnp.broadcast_to(cs, (n_steps_pad, n_exp))
    sgid = jnp.sum((cs_b <= s_iota).astype(f32), axis=1, keepdims=True)
    sgid = jnp.minimum(sgid, float(n_exp - 1))
    onehot_sg = (sgid == jax.lax.broadcasted_iota(
        jnp.int32, (n_steps_pad, n_exp), 1).astype(f32))
    onehot_sg = onehot_sg.astype(f32)

    def gath(v):
        return jnp.sum(onehot_sg * v, axis=1, keepdims=True)

    within = s_iota - (gath(cs) - gath(steps_g))
    stile = gath(tfirst) + within
    valid = s_iota < total
    n_tiles = float(2 * n_tok // tm)
    stile = jnp.where(valid, stile, n_tiles - 1)
    rs = jnp.where(valid, jnp.maximum(gath(offs), stile * tm), 0.0)
    re = jnp.where(valid, jnp.minimum(gath(ends), (stile + 1) * tm), 0.0)
    sgid_ref[...] = sgid.astype(jnp.int32)
    stile_ref[...] = stile.astype(jnp.int32)
    rs_ref[...] = rs.astype(jnp.int32)
    re_ref[...] = re.astype(jnp.int32)


def _dispatch_metadata(idx, n_exp, tm):
    n_tok = idx.shape[0]
    n_tiles = 2 * n_tok // tm
    n_steps = n_tiles + n_exp - 1
    n_steps_pad = ((n_steps + 7) // 8) * 8
    pos01, sgid, stile, rs, re = pl.pallas_call(
        functools.partial(_metadata_body, n_exp=n_exp, tm=tm,
                          n_steps_pad=n_steps_pad, n_tok=n_tok),
        grid=(1,),
        in_specs=[pl.BlockSpec((n_tok, 2), lambda i: (0, 0))],
        out_specs=[
            pl.BlockSpec((n_tok, 2), lambda i: (0, 0)),
            pl.BlockSpec((n_steps_pad, 1), lambda i: (0, 0)),
            pl.BlockSpec((n_steps_pad, 1), lambda i: (0, 0)),
            pl.BlockSpec((n_steps_pad, 1), lambda i: (0, 0)),
            pl.BlockSpec((n_steps_pad, 1), lambda i: (0, 0)),
        ],
        out_shape=[
            jax.ShapeDtypeStruct((n_tok, 2), jnp.int32),
            jax.ShapeDtypeStruct((n_steps_pad, 1), jnp.int32),
            jax.ShapeDtypeStruct((n_steps_pad, 1), jnp.int32),
            jax.ShapeDtypeStruct((n_steps_pad, 1), jnp.int32),
            jax.ShapeDtypeStruct((n_steps_pad, 1), jnp.int32),
        ],
    )(idx)
    return (pos01, sgid.reshape(-1), stile.reshape(-1), rs.reshape(-1),
            re.reshape(-1), n_steps_pad)


def kernel(x, W1, Wg, W2, sW1, sb1, sW2, sb2, gW, gb, router_biases):
    b, s_, emb = x.shape
    x2d = x.reshape(-1, emb)
    n = x2d.shape[0]
    n_exp = W1.shape[0]
    tm = 256

    idx, w = _router(x2d, gW, gb, router_biases)
    pos01, sgid, stile, rs, re, n_steps = _dispatch_metadata(idx, n_exp, tm)

    half = jnp.arange(2, dtype=jnp.int32)
    s_idx = (pos01.T.reshape(-1, 1) * 2 + half).reshape(1, -1)
    g_idx = (pos01.reshape(-1, 1) * 2 + half).reshape(1, -1)

    xs = _sc_scatter_rows(x2d.reshape(2 * n, emb // 2), s_idx)
    ys = _grouped_ffn(xs.reshape(2 * n, emb), W1, Wg, W2,
                      sgid, stile, rs, re, tm, n_steps)
    g = _sc_gather_rows(ys.reshape(4 * n, emb // 2), g_idx)
    g2 = g.reshape(n, 2 * emb)
    out = _shared_combine(x2d, sW1, sb1, sW2, sb2, g2, w)
    return out.reshape(b, s_, emb)


# speedup vs baseline: 2.0987x; 1.0079x over previous
import functools

import jax
import jax.numpy as jnp
from jax.experimental import pallas as pl
from jax.experimental.pallas import tpu as pltpu
from jax.experimental.pallas import tpu_sc as plsc


def _router_body(x_ref, gwt_ref, gb_ref, rb_ref, idx_ref, w_ref, *, n_exp):
    x = x_ref[...]
    logits = jnp.dot(x, gwt_ref[...], preferred_element_type=jnp.float32)
    logits = logits + gb_ref[...]
    m = jnp.max(logits, axis=-1, keepdims=True)
    p = jnp.exp(logits - m)
    probas = p / jnp.sum(p, axis=-1, keepdims=True)
    biased = probas + rb_ref[...]
    col = jax.lax.broadcasted_iota(jnp.int32, biased.shape, 1)
    i1 = jnp.argmax(biased, axis=-1, keepdims=True).astype(jnp.int32)
    biased2 = jnp.where(col == i1, -1e30, biased)
    i2 = jnp.argmax(biased2, axis=-1, keepdims=True).astype(jnp.int32)
    p1 = jnp.sum(jnp.where(col == i1, probas, 0.0), axis=-1, keepdims=True)
    p2 = jnp.sum(jnp.where(col == i2, probas, 0.0), axis=-1, keepdims=True)
    tot = p1 + p2
    idx_ref[...] = jnp.concatenate([i1, i2], axis=1)
    w_ref[...] = jnp.concatenate([p1 / tot, p2 / tot], axis=1)


def _router(x2d, gW, gb, router_biases):
    n, emb = x2d.shape
    n_exp = gW.shape[0]
    ep = 128
    tm = 512
    gwt = jnp.zeros((emb, ep), jnp.float32).at[:, :n_exp].set(gW.T)
    neg = jnp.full((1, ep), -1e30, jnp.float32)
    gbp = neg.at[0, :n_exp].set(gb)
    rbp = neg.at[0, :n_exp].set(router_biases)
    idx, w = pl.pallas_call(
        functools.partial(_router_body, n_exp=n_exp),
        grid=(n // tm,),
        in_specs=[
            pl.BlockSpec((tm, emb), lambda i: (i, 0)),
            pl.BlockSpec((emb, ep), lambda i: (0, 0)),
            pl.BlockSpec((1, ep), lambda i: (0, 0)),
            pl.BlockSpec((1, ep), lambda i: (0, 0)),
        ],
        out_specs=[
            pl.BlockSpec((tm, 2), lambda i: (i, 0)),
            pl.BlockSpec((tm, 2), lambda i: (i, 0)),
        ],
        out_shape=[
            jax.ShapeDtypeStruct((n, 2), jnp.int32),
            jax.ShapeDtypeStruct((n, 2), jnp.float32),
        ],
    )(x2d, gwt, gbp, rbp)
    return idx, w


def _sc_gather_rows(src, idx2d):
    m = idx2d.shape[1]
    d = src.shape[1]
    window = 128
    mesh = plsc.VectorSubcoreMesh(core_axis_name="core", subcore_axis_name="subcore")

    @functools.partial(
        pl.kernel,
        out_type=jax.ShapeDtypeStruct((m, d), src.dtype),
        mesh=mesh,
    )
    def gather_kernel(x_hbm, i_hbm, o_hbm):
        def body(i_vmem, o_vmem):
            pltpu.sync_copy(x_hbm.at[i_vmem.at[0]], o_vmem)

        pltpu.emit_pipeline(
            body,
            grid=(m // window,),
            in_specs=[pl.BlockSpec((1, window), lambda i: (0, i))],
            out_specs=[pl.BlockSpec((window, d), lambda i: (i, 0))],
            core_axis_name=("core", "subcore"),
            dimension_semantics=(pltpu.PARALLEL,),
        )(i_hbm, o_hbm)

    return gather_kernel(src, idx2d)


def _sc_scatter_rows(src, idx2d):
    rows, d = src.shape
    m = idx2d.shape[1]
    window = 128
    n_src_blocks = rows // window
    mesh = plsc.VectorSubcoreMesh(core_axis_name="core", subcore_axis_name="subcore")

    @functools.partial(
        pl.kernel,
        out_type=jax.ShapeDtypeStruct((m, d), src.dtype),
        mesh=mesh,
    )
    def scatter_kernel(x_hbm, i_hbm, o_hbm):
        def body(x_vmem, i_vmem):
            pltpu.sync_copy(x_vmem, o_hbm.at[i_vmem.at[0]])

        pltpu.emit_pipeline(
            body,
            grid=(m // window,),
            in_specs=[
                pl.BlockSpec((window, d), lambda i: (i % n_src_blocks, 0)),
                pl.BlockSpec((1, window), lambda i: (0, i)),
            ],
            out_specs=[],
            core_axis_name=("core", "subcore"),
            dimension_semantics=(pltpu.PARALLEL,),
        )(x_hbm, i_hbm)

    return scatter_kernel(src, idx2d)


def _grouped_body(sgid_ref, stile_ref, rs_ref, re_ref,
                  x_ref, w1_ref, wg_ref, w2_ref, o_ref,
                  w1b_ref, wgb_ref, w2b_ref, *, tm):
    s = pl.program_id(0)
    prev_gid = sgid_ref[jnp.maximum(s - 1, 0)]
    new_expert = jnp.logical_or(s == 0, sgid_ref[s] != prev_gid)

    @pl.when(new_expert)
    def _():
        w1b_ref[...] = w1_ref[0].astype(jnp.bfloat16)
        wgb_ref[...] = wg_ref[0].astype(jnp.bfloat16)
        w2b_ref[...] = w2_ref[0].astype(jnp.bfloat16)

    x = x_ref[...].astype(jnp.bfloat16)
    a = jnp.dot(x, w1b_ref[...], preferred_element_type=jnp.float32)
    b = jnp.dot(x, wgb_ref[...], preferred_element_type=jnp.float32)
    h = (a * jax.lax.logistic(a) * b).astype(jnp.bfloat16)
    o = jnp.dot(h, w2b_ref[...], preferred_element_type=jnp.float32)
    rows = stile_ref[s] * tm + jax.lax.broadcasted_iota(jnp.int32, (tm, 1), 0)
    keep = (rows >= rs_ref[s]) & (rows < re_ref[s])
    o = jnp.where(keep, o, 0.0)
    prev_tile = stile_ref[jnp.maximum(s - 1, 0)]
    first_visit = jnp.logical_or(s == 0, stile_ref[s] != prev_tile)

    @pl.when(first_visit)
    def _():
        o_ref[...] = o

    @pl.when(jnp.logical_not(first_visit))
    def _():
        o_ref[...] += o


def _grouped_ffn(xs, W1, Wg, W2, sgid, stile, rs, re, tm, n_steps):
    a_total, emb = xs.shape
    hid = W1.shape[2]
    return pl.pallas_call(
        functools.partial(_grouped_body, tm=tm),
        grid_spec=pltpu.PrefetchScalarGridSpec(
            num_scalar_prefetch=4,
            grid=(n_steps,),
            in_specs=[
                pl.BlockSpec((tm, emb), lambda s, g, t, a, b: (t[s], 0)),
                pl.BlockSpec((1, emb, hid), lambda s, g, t, a, b: (g[s], 0, 0)),
                pl.BlockSpec((1, emb, hid), lambda s, g, t, a, b: (g[s], 0, 0)),
                pl.BlockSpec((1, hid, emb), lambda s, g, t, a, b: (g[s], 0, 0)),
            ],
            out_specs=pl.BlockSpec((tm, emb), lambda s, g, t, a, b: (t[s], 0)),
            scratch_shapes=[
                pltpu.VMEM((emb, hid), jnp.bfloat16),
                pltpu.VMEM((emb, hid), jnp.bfloat16),
                pltpu.VMEM((hid, emb), jnp.bfloat16),
            ],
        ),
        out_shape=jax.ShapeDtypeStruct((a_total, emb), jnp.float32),
    )(sgid, stile, rs, re, xs, W1, Wg, W2)


def _shared_body(x_ref, w1_ref, b1_ref, w2_ref, b2_ref, g_ref, w_ref, o_ref,
                 w1b_ref, w2b_ref, *, emb):
    @pl.when(pl.program_id(0) == 0)
    def _():
        w1b_ref[...] = w1_ref[...].astype(jnp.bfloat16)
        w2b_ref[...] = w2_ref[...].astype(jnp.bfloat16)

    x = x_ref[...].astype(jnp.bfloat16)
    h = jnp.dot(x, w1b_ref[...], preferred_element_type=jnp.float32) + b1_ref[...]
    h = (h * jax.lax.logistic(h)).astype(jnp.bfloat16)
    o = jnp.dot(h, w2b_ref[...], preferred_element_type=jnp.float32) + b2_ref[...]
    g = g_ref[...]
    w = w_ref[...]
    o_ref[...] = o + w[:, 0:1] * g[:, :emb] + w[:, 1:2] * g[:, emb:]


def _shared_combine(x2d, sW1, sb1, sW2, sb2, g2, w):
    n, emb = x2d.shape
    hid = sW1.shape[2]
    h2 = 2 * hid
    tm = 256
    w1c = jnp.concatenate([sW1[0], sW1[1]], axis=1)
    b1c = sb1.reshape(1, h2)
    w2c = sW2.reshape(h2, emb)
    b2c = (sb2[0] + sb2[1]).reshape(1, emb)
    return pl.pallas_call(
        functools.partial(_shared_body, emb=emb),
        grid=(n // tm,),
        in_specs=[
            pl.BlockSpec((tm, emb), lambda i: (i, 0)),
            pl.BlockSpec((emb, h2), lambda i: (0, 0)),
            pl.BlockSpec((1, h2), lambda i: (0, 0)),
            pl.BlockSpec((h2, emb), lambda i: (0, 0)),
            pl.BlockSpec((1, emb), lambda i: (0, 0)),
            pl.BlockSpec((tm, 2 * emb), lambda i: (i, 0)),
            pl.BlockSpec((tm, 2), lambda i: (i, 0)),
        ],
        out_specs=pl.BlockSpec((tm, emb), lambda i: (i, 0)),
        out_shape=jax.ShapeDtypeStruct((n, emb), jnp.float32),
        scratch_shapes=[
            pltpu.VMEM((emb, h2), jnp.bfloat16),
            pltpu.VMEM((h2, emb), jnp.bfloat16),
        ],
    )(x2d, w1c, b1c, w2c, b2c, g2, w)


def _metadata_body(idx_ref, pos_ref, sgid_ref, stile_ref, rs_ref, re_ref,
                   *, n_exp, tm, n_steps_pad, n_tok):
    f32 = jnp.float32
    idx = idx_ref[...]
    e_iota = jax.lax.broadcasted_iota(jnp.int32, (n_tok, n_exp), 1)
    oh0 = (idx[:, 0:1] == e_iota).astype(f32)
    oh1 = (idx[:, 1:2] == e_iota).astype(f32)

    ohb = jnp.concatenate([oh0, oh1], axis=1)
    acc = ohb
    sh = 1
    while sh < n_tok:
        acc = acc + jnp.concatenate(
            [jnp.zeros((sh, 2 * n_exp), f32), acc[: n_tok - sh, :]], axis=0)
        sh *= 2
    tot = acc[n_tok - 1:n_tok, :]
    tot0 = tot[:, :n_exp]
    tot1 = tot[:, n_exp:]
    ex0 = acc[:, :n_exp] - oh0
    ex1 = acc[:, n_exp:] - oh1
    counts = tot0 + tot1

    def _lane_prefix_excl(v):
        acc = v
        sh = 1
        while sh < n_exp:
            acc = acc + jnp.concatenate(
                [jnp.zeros((1, sh), f32), acc[:, : n_exp - sh]], axis=1)
            sh *= 2
        return acc - v

    offs = _lane_prefix_excl(counts)
    pos0 = jnp.sum(jnp.where(oh0 > 0.5, ex0 + offs, 0.0),
                   axis=1, keepdims=True)
    pos1 = jnp.sum(jnp.where(oh1 > 0.5, ex1 + offs + tot0, 0.0),
                   axis=1, keepdims=True)
    pos_ref[...] = jnp.round(
        jnp.concatenate([pos0, pos1], axis=1)).astype(jnp.int32)

    ends = offs + counts
    tfirst = jnp.floor(offs / tm)
    tlast = jnp.where(counts > 0, jnp.floor((ends - 1) / tm), tfirst)
    steps_g = jnp.where(counts > 0, tlast - tfirst + 1, 0.0)
    cs = _lane_prefix_excl(steps_g) + steps_g
    total = jnp.sum(steps_g)
    s_iota = jax.lax.broadcasted_iota(
        jnp.int32, (n_steps_pad, 1), 0).astype(f32)
    cs_b = jnp.broadcast_to(cs, (n_steps_pad, n_exp))
    sgid = jnp.sum((cs_b <= s_iota).astype(f32), axis=1, keepdims=True)
    sgid = jnp.minimum(sgid, float(n_exp - 1))
    onehot_sg = (sgid == jax.lax.broadcasted_iota(
        jnp.int32, (n_steps_pad, n_exp), 1).astype(f32))
    onehot_sg = onehot_sg.astype(f32)

    def gath(v):
        return jnp.sum(onehot_sg * v, axis=1, keepdims=True)

    within = s_iota - (gath(cs) - gath(steps_g))
    stile = gath(tfirst) + within
    valid = s_iota < total
    n_tiles = float(2 * n_tok // tm)
    stile = jnp.where(valid, stile, n_tiles - 1)
    rs = jnp.where(valid, jnp.maximum(gath(offs), stile * tm), 0.0)
    re = jnp.where(valid, jnp.minimum(gath(ends), (stile + 1) * tm), 0.0)
    sgid_ref[...] = sgid.astype(jnp.int32)
    stile_ref[...] = stile.astype(jnp.int32)
    rs_ref[...] = rs.astype(jnp.int32)
    re_ref[...] = re.astype(jnp.int32)


def _dispatch_metadata(idx, n_exp, tm):
    n_tok = idx.shape[0]
    n_tiles = 2 * n_tok // tm
    n_steps = n_tiles + n_exp - 1
    n_steps_pad = ((n_steps + 7) // 8) * 8
    pos01, sgid, stile, rs, re = pl.pallas_call(
        functools.partial(_metadata_body, n_exp=n_exp, tm=tm,
                          n_steps_pad=n_steps_pad, n_tok=n_tok),
        grid=(1,),
        in_specs=[pl.BlockSpec((n_tok, 2), lambda i: (0, 0))],
        out_specs=[
            pl.BlockSpec((n_tok, 2), lambda i: (0, 0)),
            pl.BlockSpec((n_steps_pad, 1), lambda i: (0, 0)),
            pl.BlockSpec((n_steps_pad, 1), lambda i: (0, 0)),
            pl.BlockSpec((n_steps_pad, 1), lambda i: (0, 0)),
            pl.BlockSpec((n_steps_pad, 1), lambda i: (0, 0)),
        ],
        out_shape=[
            jax.ShapeDtypeStruct((n_tok, 2), jnp.int32),
            jax.ShapeDtypeStruct((n_steps_pad, 1), jnp.int32),
            jax.ShapeDtypeStruct((n_steps_pad, 1), jnp.int32),
            jax.ShapeDtypeStruct((n_steps_pad, 1), jnp.int32),
            jax.ShapeDtypeStruct((n_steps_pad, 1), jnp.int32),
        ],
    )(idx)
    return (pos01, sgid.reshape(-1), stile.reshape(-1), rs.reshape(-1),
            re.reshape(-1), n_steps_pad)


def kernel(x, W1, Wg, W2, sW1, sb1, sW2, sb2, gW, gb, router_biases):
    b, s_, emb = x.shape
    x2d = x.reshape(-1, emb)
    n = x2d.shape[0]
    n_exp = W1.shape[0]
    tm = 512

    idx, w = _router(x2d, gW, gb, router_biases)
    pos01, sgid, stile, rs, re, n_steps = _dispatch_metadata(idx, n_exp, tm)

    half = jnp.arange(2, dtype=jnp.int32)
    s_idx = (pos01.T.reshape(-1, 1) * 2 + half).reshape(1, -1)
    g_idx = (pos01.reshape(-1, 1) * 2 + half).reshape(1, -1)

    xs = _sc_scatter_rows(x2d.reshape(2 * n, emb // 2), s_idx)
    ys = _grouped_ffn(xs.reshape(2 * n, emb), W1, Wg, W2,
                      sgid, stile, rs, re, tm, n_steps)
    g = _sc_gather_rows(ys.reshape(4 * n, emb // 2), g_idx)
    g2 = g.reshape(n, 2 * emb)
    out = _shared_combine(x2d, sW1, sb1, sW2, sb2, g2, w)
    return out.reshape(b, s_, emb)


# speedup vs baseline: 2.1251x; 1.0126x over previous
import functools

import jax
import jax.numpy as jnp
from jax.experimental import pallas as pl
from jax.experimental.pallas import tpu as pltpu
from jax.experimental.pallas import tpu_sc as plsc


def _router_body(x_ref, gwt_ref, gb_ref, rb_ref, idx_ref, w_ref, *, n_exp):
    x = x_ref[...]
    logits = jnp.dot(x, gwt_ref[...], preferred_element_type=jnp.float32)
    logits = logits + gb_ref[...]
    m = jnp.max(logits, axis=-1, keepdims=True)
    p = jnp.exp(logits - m)
    probas = p / jnp.sum(p, axis=-1, keepdims=True)
    biased = probas + rb_ref[...]
    col = jax.lax.broadcasted_iota(jnp.int32, biased.shape, 1)
    i1 = jnp.argmax(biased, axis=-1, keepdims=True).astype(jnp.int32)
    biased2 = jnp.where(col == i1, -1e30, biased)
    i2 = jnp.argmax(biased2, axis=-1, keepdims=True).astype(jnp.int32)
    p1 = jnp.sum(jnp.where(col == i1, probas, 0.0), axis=-1, keepdims=True)
    p2 = jnp.sum(jnp.where(col == i2, probas, 0.0), axis=-1, keepdims=True)
    tot = p1 + p2
    idx_ref[...] = jnp.concatenate([i1, i2], axis=1)
    w_ref[...] = jnp.concatenate([p1 / tot, p2 / tot], axis=1)


def _router(x2d, gW, gb, router_biases):
    n, emb = x2d.shape
    n_exp = gW.shape[0]
    ep = 128
    tm = 512
    gwt = jnp.zeros((emb, ep), jnp.float32).at[:, :n_exp].set(gW.T)
    neg = jnp.full((1, ep), -1e30, jnp.float32)
    gbp = neg.at[0, :n_exp].set(gb)
    rbp = neg.at[0, :n_exp].set(router_biases)
    idx, w = pl.pallas_call(
        functools.partial(_router_body, n_exp=n_exp),
        grid=(n // tm,),
        in_specs=[
            pl.BlockSpec((tm, emb), lambda i: (i, 0)),
            pl.BlockSpec((emb, ep), lambda i: (0, 0)),
            pl.BlockSpec((1, ep), lambda i: (0, 0)),
            pl.BlockSpec((1, ep), lambda i: (0, 0)),
        ],
        out_specs=[
            pl.BlockSpec((tm, 2), lambda i: (i, 0)),
            pl.BlockSpec((tm, 2), lambda i: (i, 0)),
        ],
        out_shape=[
            jax.ShapeDtypeStruct((n, 2), jnp.int32),
            jax.ShapeDtypeStruct((n, 2), jnp.float32),
        ],
    )(x2d, gwt, gbp, rbp)
    return idx, w


def _sc_gather_rows(src, idx2d):
    m = idx2d.shape[1]
    d = src.shape[1]
    window = 128
    mesh = plsc.VectorSubcoreMesh(core_axis_name="core", subcore_axis_name="subcore")

    @functools.partial(
        pl.kernel,
        out_type=jax.ShapeDtypeStruct((m, d), src.dtype),
        mesh=mesh,
    )
    def gather_kernel(x_hbm, i_hbm, o_hbm):
        def body(i_vmem, o_vmem):
            pltpu.sync_copy(x_hbm.at[i_vmem.at[0]], o_vmem)

        pltpu.emit_pipeline(
            body,
            grid=(m // window,),
            in_specs=[pl.BlockSpec((1, window), lambda i: (0, i))],
            out_specs=[pl.BlockSpec((window, d), lambda i: (i, 0))],
            core_axis_name=("core", "subcore"),
            dimension_semantics=(pltpu.PARALLEL,),
        )(i_hbm, o_hbm)

    return gather_kernel(src, idx2d)


def _sc_scatter_rows(src, idx2d):
    rows, d = src.shape
    m = idx2d.shape[1]
    window = 128
    n_src_blocks = rows // window
    mesh = plsc.VectorSubcoreMesh(core_axis_name="core", subcore_axis_name="subcore")

    @functools.partial(
        pl.kernel,
        out_type=jax.ShapeDtypeStruct((m, d), src.dtype),
        mesh=mesh,
    )
    def scatter_kernel(x_hbm, i_hbm, o_hbm):
        def body(x_vmem, i_vmem):
            pltpu.sync_copy(x_vmem, o_hbm.at[i_vmem.at[0]])

        pltpu.emit_pipeline(
            body,
            grid=(m // window,),
            in_specs=[
                pl.BlockSpec((window, d), lambda i: (i % n_src_blocks, 0)),
                pl.BlockSpec((1, window), lambda i: (0, i)),
            ],
            out_specs=[],
            core_axis_name=("core", "subcore"),
            dimension_semantics=(pltpu.PARALLEL,),
        )(x_hbm, i_hbm)

    return scatter_kernel(src, idx2d)


def _grouped_body(sgid_ref, stile_ref, rs_ref, re_ref,
                  x_ref, w1_ref, wg_ref, w2_ref, o_ref,
                  w1b_ref, wgb_ref, w2b_ref, *, tm):
    s = pl.program_id(0)
    prev_gid = sgid_ref[jnp.maximum(s - 1, 0)]
    new_expert = jnp.logical_or(s == 0, sgid_ref[s] != prev_gid)

    @pl.when(new_expert)
    def _():
        w1b_ref[...] = w1_ref[0].astype(jnp.bfloat16)
        wgb_ref[...] = wg_ref[0].astype(jnp.bfloat16)
        w2b_ref[...] = w2_ref[0].astype(jnp.bfloat16)

    x = x_ref[...].astype(jnp.bfloat16)
    a = jnp.dot(x, w1b_ref[...], preferred_element_type=jnp.float32)
    b = jnp.dot(x, wgb_ref[...], preferred_element_type=jnp.float32)
    h = (a * jax.lax.logistic(a) * b).astype(jnp.bfloat16)
    o = jnp.dot(h, w2b_ref[...], preferred_element_type=jnp.float32)
    rows = stile_ref[s] * tm + jax.lax.broadcasted_iota(jnp.int32, (tm, 1), 0)
    keep = (rows >= rs_ref[s]) & (rows < re_ref[s])
    o = jnp.where(keep, o, 0.0)
    prev_tile = stile_ref[jnp.maximum(s - 1, 0)]
    first_visit = jnp.logical_or(s == 0, stile_ref[s] != prev_tile)
    o_ref[...] = jnp.where(first_visit, o, o_ref[...] + o)


def _grouped_ffn(xs, W1, Wg, W2, sgid, stile, rs, re, tm, n_steps):
    a_total, emb = xs.shape
    hid = W1.shape[2]
    return pl.pallas_call(
        functools.partial(_grouped_body, tm=tm),
        grid_spec=pltpu.PrefetchScalarGridSpec(
            num_scalar_prefetch=4,
            grid=(n_steps,),
            in_specs=[
                pl.BlockSpec((tm, emb), lambda s, g, t, a, b: (t[s], 0)),
                pl.BlockSpec((1, emb, hid), lambda s, g, t, a, b: (g[s], 0, 0)),
                pl.BlockSpec((1, emb, hid), lambda s, g, t, a, b: (g[s], 0, 0)),
                pl.BlockSpec((1, hid, emb), lambda s, g, t, a, b: (g[s], 0, 0)),
            ],
            out_specs=pl.BlockSpec((tm, emb), lambda s, g, t, a, b: (t[s], 0)),
            scratch_shapes=[
                pltpu.VMEM((emb, hid), jnp.bfloat16),
                pltpu.VMEM((emb, hid), jnp.bfloat16),
                pltpu.VMEM((hid, emb), jnp.bfloat16),
            ],
        ),
        out_shape=jax.ShapeDtypeStruct((a_total, emb), jnp.float32),
    )(sgid, stile, rs, re, xs, W1, Wg, W2)


def _shared_body(x_ref, w1_ref, b1_ref, w2_ref, b2_ref, g_ref, w_ref, o_ref,
                 w1b_ref, w2b_ref, *, emb):
    @pl.when(pl.program_id(0) == 0)
    def _():
        w1b_ref[...] = w1_ref[...].astype(jnp.bfloat16)
        w2b_ref[...] = w2_ref[...].astype(jnp.bfloat16)

    x = x_ref[...].astype(jnp.bfloat16)
    h = jnp.dot(x, w1b_ref[...], preferred_element_type=jnp.float32) + b1_ref[...]
    h = (h * jax.lax.logistic(h)).astype(jnp.bfloat16)
    o = jnp.dot(h, w2b_ref[...], preferred_element_type=jnp.float32) + b2_ref[...]
    g = g_ref[...]
    w = w_ref[...]
    o_ref[...] = o + w[:, 0:1] * g[:, :emb] + w[:, 1:2] * g[:, emb:]


def _shared_combine(x2d, sW1, sb1, sW2, sb2, g2, w):
    n, emb = x2d.shape
    hid = sW1.shape[2]
    h2 = 2 * hid
    tm = 256
    w1c = jnp.concatenate([sW1[0], sW1[1]], axis=1)
    b1c = sb1.reshape(1, h2)
    w2c = sW2.reshape(h2, emb)
    b2c = (sb2[0] + sb2[1]).reshape(1, emb)
    return pl.pallas_call(
        functools.partial(_shared_body, emb=emb),
        grid=(n // tm,),
        in_specs=[
            pl.BlockSpec((tm, emb), lambda i: (i, 0)),
            pl.BlockSpec((emb, h2), lambda i: (0, 0)),
            pl.BlockSpec((1, h2), lambda i: (0, 0)),
            pl.BlockSpec((h2, emb), lambda i: (0, 0)),
            pl.BlockSpec((1, emb), lambda i: (0, 0)),
            pl.BlockSpec((tm, 2 * emb), lambda i: (i, 0)),
            pl.BlockSpec((tm, 2), lambda i: (i, 0)),
        ],
        out_specs=pl.BlockSpec((tm, emb), lambda i: (i, 0)),
        out_shape=jax.ShapeDtypeStruct((n, emb), jnp.float32),
        scratch_shapes=[
            pltpu.VMEM((emb, h2), jnp.bfloat16),
            pltpu.VMEM((h2, emb), jnp.bfloat16),
        ],
    )(x2d, w1c, b1c, w2c, b2c, g2, w)


def _metadata_body(idx_ref, pos_ref, sgid_ref, stile_ref, rs_ref, re_ref,
                   *, n_exp, tm, n_steps_pad, n_tok):
    f32 = jnp.float32
    idx = idx_ref[...]
    e_iota = jax.lax.broadcasted_iota(jnp.int32, (n_tok, n_exp), 1)
    oh0 = (idx[:, 0:1] == e_iota).astype(f32)
    oh1 = (idx[:, 1:2] == e_iota).astype(f32)

    ohb = jnp.concatenate([oh0, oh1], axis=1)
    acc = ohb
    sh = 1
    while sh < n_tok:
        acc = acc + jnp.concatenate(
            [jnp.zeros((sh, 2 * n_exp), f32), acc[: n_tok - sh, :]], axis=0)
        sh *= 2
    tot = acc[n_tok - 1:n_tok, :]
    tot0 = tot[:, :n_exp]
    tot1 = tot[:, n_exp:]
    ex0 = acc[:, :n_exp] - oh0
    ex1 = acc[:, n_exp:] - oh1
    counts = tot0 + tot1

    def _lane_prefix_excl(v):
        acc = v
        sh = 1
        while sh < n_exp:
            acc = acc + jnp.concatenate(
                [jnp.zeros((1, sh), f32), acc[:, : n_exp - sh]], axis=1)
            sh *= 2
        return acc - v

    offs = _lane_prefix_excl(counts)
    pos0 = jnp.sum(jnp.where(oh0 > 0.5, ex0 + offs, 0.0),
                   axis=1, keepdims=True)
    pos1 = jnp.sum(jnp.where(oh1 > 0.5, ex1 + offs + tot0, 0.0),
                   axis=1, keepdims=True)
    pos_ref[...] = jnp.round(
        jnp.concatenate([pos0, pos1], axis=1)).astype(jnp.int32)

    ends = offs + counts
    tfirst = jnp.floor(offs / tm)
    tlast = jnp.where(counts > 0, jnp.floor((ends - 1) / tm), tfirst)
    steps_g = jnp.where(counts > 0, tlast - tfirst + 1, 0.0)
    cs = _lane_prefix_excl(steps_g) + steps_g
    total = jnp.sum(steps_g)
    s_iota = jax.lax.broadcasted_iota(
        jnp.int32, (n_steps_pad, 1), 0).astype(f32)
    cs_b = jnp.broadcast_to(cs, (n_steps_pad, n_exp))
    sgid = jnp.sum((cs_b <= s_iota).astype(f32), axis=1, keepdims=True)
    sgid = jnp.minimum(sgid, float(n_exp - 1))
    onehot_sg = (sgid == jax.lax.broadcasted_iota(
        jnp.int32, (n_steps_pad, n_exp), 1).astype(f32))
    onehot_sg = onehot_sg.astype(f32)

    def gath(v):
        return jnp.sum(onehot_sg * v, axis=1, keepdims=True)

    within = s_iota - (gath(cs) - gath(steps_g))
    stile = gath(tfirst) + within
    valid = s_iota < total
    n_tiles = float(2 * n_tok // tm)
    stile = jnp.where(valid, stile, n_tiles - 1)
    rs = jnp.where(valid, jnp.maximum(gath(offs), stile * tm), 0.0)
    re = jnp.where(valid, jnp.minimum(gath(ends), (stile + 1) * tm), 0.0)
    sgid_ref[...] = sgid.astype(jnp.int32)
    stile_ref[...] = stile.astype(jnp.int32)
    rs_ref[...] = rs.astype(jnp.int32)
    re_ref[...] = re.astype(jnp.int32)


def _dispatch_metadata(idx, n_exp, tm):
    n_tok = idx.shape[0]
    n_tiles = 2 * n_tok // tm
    n_steps = n_tiles + n_exp - 1
    n_steps_pad = ((n_steps + 7) // 8) * 8
    pos01, sgid, stile, rs, re = pl.pallas_call(
        functools.partial(_metadata_body, n_exp=n_exp, tm=tm,
                          n_steps_pad=n_steps_pad, n_tok=n_tok),
        grid=(1,),
        in_specs=[pl.BlockSpec((n_tok, 2), lambda i: (0, 0))],
        out_specs=[
            pl.BlockSpec((n_tok, 2), lambda i: (0, 0)),
            pl.BlockSpec((n_steps_pad, 1), lambda i: (0, 0)),
            pl.BlockSpec((n_steps_pad, 1), lambda i: (0, 0)),
            pl.BlockSpec((n_steps_pad, 1), lambda i: (0, 0)),
            pl.BlockSpec((n_steps_pad, 1), lambda i: (0, 0)),
        ],
        out_shape=[
            jax.ShapeDtypeStruct((n_tok, 2), jnp.int32),
            jax.ShapeDtypeStruct((n_steps_pad, 1), jnp.int32),
            jax.ShapeDtypeStruct((n_steps_pad, 1), jnp.int32),
            jax.ShapeDtypeStruct((n_steps_pad, 1), jnp.int32),
            jax.ShapeDtypeStruct((n_steps_pad, 1), jnp.int32),
        ],
    )(idx)
    return (pos01, sgid.reshape(-1), stile.reshape(-1), rs.reshape(-1),
            re.reshape(-1), n_steps_pad)


def kernel(x, W1, Wg, W2, sW1, sb1, sW2, sb2, gW, gb, router_biases):
    b, s_, emb = x.shape
    x2d = x.reshape(-1, emb)
    n = x2d.shape[0]
    n_exp = W1.shape[0]
    tm = 512

    idx, w = _router(x2d, gW, gb, router_biases)
    pos01, sgid, stile, rs, re, n_steps = _dispatch_metadata(idx, n_exp, tm)

    half = jnp.arange(2, dtype=jnp.int32)
    s_idx = (pos01.T.reshape(-1, 1) * 2 + half).reshape(1, -1)
    g_idx = (pos01.reshape(-1, 1) * 2 + half).reshape(1, -1)

    xs = _sc_scatter_rows(x2d.reshape(2 * n, emb // 2), s_idx)
    ys = _grouped_ffn(xs.reshape(2 * n, emb), W1, Wg, W2,
                      sgid, stile, rs, re, tm, n_steps)
    g = _sc_gather_rows(ys.reshape(4 * n, emb // 2), g_idx)
    g2 = g.reshape(n, 2 * emb)
    out = _shared_combine(x2d, sW1, sb1, sW2, sb2, g2, w)
    return out.reshape(b, s_, emb)
